# baseline (speedup 1.0000x reference)
import jax
import jax.numpy as jnp
from jax import lax
from jax.experimental import pallas as pl
from jax.experimental.pallas import tpu as pltpu


def _chunks(n, nc):
    base = ((n // nc) // 8) * 8
    offs = [i * base for i in range(nc)]
    sizes = [base] * (nc - 1) + [n - base * (nc - 1)]
    return list(zip(offs, sizes))


def _body(n, d, nc, x_ref, w_ref, xout_ref, score_ref, xs_ref, ob_ref,
          in_sems, out_sems):
    chunks = _chunks(n, nc)
    w = w_ref[...]
    w2 = w * lax.rsqrt(jnp.sum(w * w))
    w2t = lax.transpose(w2, (1, 0))
    wb = lax.broadcast_in_dim(w2t, (d, d), (0, 1))

    for c, (off, sz) in enumerate(chunks):
        pltpu.make_async_copy(
            x_ref.at[pl.ds(off, sz), :], xs_ref.at[pl.ds(off, sz), :],
            in_sems.at[c],
        ).start()

    srows = []
    for c, (off, sz) in enumerate(chunks):
        pltpu.make_async_copy(
            x_ref.at[pl.ds(off, sz), :], xs_ref.at[pl.ds(off, sz), :],
            in_sems.at[c],
        ).wait()
        srows.append(lax.dot_general(
            w2, xs_ref[pl.ds(off, sz), :], (((1,), (1,)), ((), ())),
            preferred_element_type=jnp.float32,
        ))
    s_row = lax.concatenate(srows, 1)
    cm = jnp.sum(s_row) / n
    score_ref[...] = jnp.tanh(s_row - cm)

    for c, (off, sz) in enumerate(chunks):
        if c >= 2:
            poff, psz = chunks[c - 2]
            pltpu.make_async_copy(
                ob_ref.at[c % 2, pl.ds(0, psz), :],
                xout_ref.at[pl.ds(poff, psz), :], out_sems.at[c - 2],
            ).wait()
        xv = xs_ref[pl.ds(off, sz), :]
        sb = lax.dot_general(
            xv, wb, (((1,), (0,)), ((), ())), preferred_element_type=jnp.float32
        )
        ob_ref[c % 2, pl.ds(0, sz), :] = xv * jnp.tanh(sb - cm)
        pltpu.make_async_copy(
            ob_ref.at[c % 2, pl.ds(0, sz), :],
            xout_ref.at[pl.ds(off, sz), :], out_sems.at[c],
        ).start()
    for c in range(max(len(chunks) - 2, 0), len(chunks)):
        poff, psz = chunks[c]
        pltpu.make_async_copy(
            ob_ref.at[c % 2, pl.ds(0, psz), :],
            xout_ref.at[pl.ds(poff, psz), :], out_sems.at[c],
        ).wait()


def kernel(x, edge_index, weight):
    n, d = x.shape
    nc = 3
    max_sz = max(sz for _, sz in _chunks(n, nc))

    def body(*refs):
        _body(n, d, nc, *refs)

    x_out, score = pl.pallas_call(
        body,
        in_specs=[
            pl.BlockSpec(memory_space=pl.ANY),
            pl.BlockSpec((1, d), lambda: (0, 0)),
        ],
        out_specs=[
            pl.BlockSpec(memory_space=pl.ANY),
            pl.BlockSpec((1, n), lambda: (0, 0)),
        ],
        out_shape=(
            jax.ShapeDtypeStruct((n, d), x.dtype),
            jax.ShapeDtypeStruct((1, n), x.dtype),
        ),
        scratch_shapes=[
            pltpu.VMEM((n, d), jnp.float32),
            pltpu.VMEM((2, max_sz, d), jnp.float32),
            pltpu.SemaphoreType.DMA((nc,)),
            pltpu.SemaphoreType.DMA((nc,)),
        ],
    )(x, weight)
    return x_out, score
